# baseline (speedup 1.0000x reference)
import functools

import numpy as np
import jax
import jax.numpy as jnp
from jax import lax
from jax.experimental import pallas as pl
from jax.experimental.pallas import tpu as pltpu
from jax.experimental.pallas import tpu_sc as plsc

R_CUT = 5.0
RBF_DIM = 20
NODE = 128
N = 10000
NPAD = 10240
E = 320000
NPASS = 4
K = 16
NBUF = 4
TILES = 32
EPT = E // TILES
CPT = EPT // K
ROWS_PT = NPAD // 16

_PERM = np.concatenate(
    [np.arange(128 * b + 32 * p, 128 * b + 32 * p + 32)
     for p in range(NPASS) for b in range(3)]
)


_BN1 = 1000


def _k1_body(ns_ref, nv_ref, w1_ref, b1_ref, w2p_ref, b2p_ref, g_ref):
    h = jnp.dot(ns_ref[...], w1_ref[...], preferred_element_type=jnp.float32)
    h = h + b1_ref[...]
    h = h * jax.nn.sigmoid(h)
    a = jnp.dot(h, w2p_ref[...], preferred_element_type=jnp.float32)
    a = a + b2p_ref[...]
    nv = nv_ref[...]
    for p in range(NPASS):
        g_ref[p, :, 0:96] = a[:, 96 * p:96 * p + 96]
        for d in range(3):
            g_ref[p, :, 96 + 32 * d:128 + 32 * d] = nv[:, d, 32 * p:32 * p + 32]
        g_ref[p, :, 192:256] = jnp.zeros((_BN1, 64), jnp.float32)


def _build_node_table(ns, nv, W1, b1, W2p, b2p):
    return pl.pallas_call(
        _k1_body,
        grid=(N // _BN1,),
        in_specs=[
            pl.BlockSpec((_BN1, NODE), lambda i: (i, 0)),
            pl.BlockSpec((_BN1, 3, NODE), lambda i: (i, 0, 0)),
            pl.BlockSpec((NODE, NODE), lambda i: (0, 0)),
            pl.BlockSpec((1, NODE), lambda i: (0, 0)),
            pl.BlockSpec((NODE, 3 * NODE), lambda i: (0, 0)),
            pl.BlockSpec((1, 3 * NODE), lambda i: (0, 0)),
        ],
        out_specs=pl.BlockSpec((NPASS, _BN1, 256), lambda i: (0, i, 0)),
        out_shape=jax.ShapeDtypeStruct((NPASS, NPAD, 256), jnp.float32),
    )(ns, nv, W1, b1, W2p, b2p)


_ROWS_A = 2500
_EROW = E // 128


def _k2a_body(d_ref, f_ref):
    d = d_ref[...]
    x = (jnp.pi / R_CUT) * d
    s1 = jnp.sin(x)
    c1 = jnp.cos(x)
    cut = 0.5 * c1 + 0.5
    g = cut / d
    two_c = 2.0 * c1
    s_prev = jnp.zeros_like(s1)
    s_cur = s1
    for k in range(RBF_DIM):
        f_ref[k, :, :] = s_cur * g
        s_next = two_c * s_cur - s_prev
        s_prev = s_cur
        s_cur = s_next
    f_ref[RBF_DIM, :, :] = cut
    z = jnp.zeros_like(s1)
    f_ref[RBF_DIM + 1, :, :] = z
    f_ref[RBF_DIM + 2, :, :] = z
    f_ref[RBF_DIM + 3, :, :] = z


def _build_basis(dist2d):
    return pl.pallas_call(
        _k2a_body,
        grid=(_EROW // _ROWS_A,),
        in_specs=[pl.BlockSpec((_ROWS_A, 128), lambda i: (i, 0))],
        out_specs=pl.BlockSpec((24, _ROWS_A, 128), lambda i: (0, i, 0)),
        out_shape=jax.ShapeDtypeStruct((24, _EROW, 128), jnp.float32),
    )(dist2d)


_BE2 = 3200


def _k2b_body(f_ref, sp_ref, wext_ref, out_ref):
    f = f_ref[...]
    sp = sp_ref[...]
    spz = jnp.concatenate(
        [jnp.zeros((_BE2, 96), jnp.float32), sp,
         jnp.zeros((_BE2, 29), jnp.float32)], axis=1)
    for p in range(NPASS):
        tp = lax.dot_general(
            f, wext_ref[p],
            dimension_numbers=(((0,), (0,)), ((), ())),
            preferred_element_type=jnp.float32)
        out_ref[p, :, :] = tp + spz


def _build_edge_filter(F, edges_sense, Wext):
    return pl.pallas_call(
        _k2b_body,
        grid=(E // _BE2,),
        in_specs=[
            pl.BlockSpec((24, _BE2), lambda i: (0, i)),
            pl.BlockSpec((_BE2, 3), lambda i: (i, 0)),
            pl.BlockSpec((NPASS, 24, 128), lambda i: (0, 0, 0)),
        ],
        out_specs=pl.BlockSpec((NPASS, _BE2, 128), lambda i: (0, i, 0)),
        out_shape=jax.ShapeDtypeStruct((NPASS, E, 128), jnp.float32),
    )(F, edges_sense, Wext)


_sc_mesh = plsc.VectorSubcoreMesh(
    core_axis_name="c", subcore_axis_name="s", num_cores=2)


@functools.partial(
    pl.kernel,
    out_type=jax.ShapeDtypeStruct((2, NPASS, NPAD, 128), jnp.float32),
    mesh=_sc_mesh,
    compiler_params=pltpu.CompilerParams(needs_layout_passes=False),
    scratch_types=(
        [pltpu.VMEM((K, 256), jnp.float32) for _ in range(NBUF)]
        + [pltpu.VMEM((K, 128), jnp.float32) for _ in range(NBUF)]
        + [pltpu.VMEM((2 * K,), jnp.int32) for _ in range(NBUF)]
        + [pltpu.VMEM((K, 128), jnp.float32) for _ in range(NBUF)]
        + [pltpu.VMEM_SHARED((NPAD, 128), jnp.float32)]
        + [pltpu.SemaphoreType.DMA for _ in range(3 * NBUF)]
    ),
)
def _sc_edges(g_hbm, d_hbm, gr_hbm, out_hbm, *scr):
    g_vs = scr[0:NBUF]
    d_vs = scr[NBUF:2 * NBUF]
    r_vs = scr[2 * NBUF:3 * NBUF]
    m_vs = scr[3 * NBUF:4 * NBUF]
    slab = scr[4 * NBUF]
    gsems = scr[4 * NBUF + 1:4 * NBUF + 1 + NBUF]
    dsems = scr[4 * NBUF + 1 + NBUF:4 * NBUF + 1 + 2 * NBUF]
    ssems = scr[4 * NBUF + 1 + 2 * NBUF:4 * NBUF + 1 + 3 * NBUF]

    cid = lax.axis_index("c")
    sid = lax.axis_index("s")
    wid = cid * 16 + sid
    row0 = sid * ROWS_PT

    iota2 = lax.iota(jnp.int32, 16) * 2

    def _issue_lin(jc, b, p):
        e0 = wid * EPT + jc * K
        pltpu.async_copy(gr_hbm.at[pl.ds(2 * e0, 2 * K)], r_vs[b], dsems[b])
        pltpu.async_copy(d_hbm.at[p, pl.ds(e0, K)], d_vs[b], dsems[b])

    def _issue_g(jc, b, p):
        dst = plsc.load_gather(r_vs[b], [iota2 + 1])
        pltpu.async_copy(g_hbm.at[dst + p * NPAD], g_vs[b], gsems[b])

    def _wait_lin(b):
        pltpu.make_async_copy(gr_hbm.at[pl.ds(0, 2 * K)], r_vs[b], dsems[b]).wait()
        pltpu.make_async_copy(d_hbm.at[0, pl.ds(0, K)], d_vs[b], dsems[b]).wait()

    def _wait_g(b):
        pltpu.make_async_copy(g_hbm.at[pl.ds(0, K)], g_vs[b], gsems[b]).wait()

    def _wait_sc(b):
        pltpu.make_async_copy(m_vs[b], slab.at[pl.ds(0, K)], ssems[b]).wait()

    def _compute(b):
        g_v = g_vs[b]
        d_v = d_vs[b]
        msg_v = m_vs[b]
        for e in range(K):
            sv = d_v[e, pl.ds(96, 16)]
            s0 = sv[0]
            s1 = sv[1]
            s2 = sv[2]
            for h in range(2):
                o = 16 * h
                a0 = g_v[e, pl.ds(o, 16)]
                a1 = g_v[e, pl.ds(32 + o, 16)]
                a2 = g_v[e, pl.ds(64 + o, 16)]
                dd0 = d_v[e, pl.ds(o, 16)]
                dd1 = d_v[e, pl.ds(32 + o, 16)]
                dd2 = d_v[e, pl.ds(64 + o, 16)]
                gate = a0 * dd0
                dirv = a2 * dd2
                msg_v[e, pl.ds(o, 16)] = a1 * dd1
                v0 = g_v[e, pl.ds(96 + o, 16)]
                v1 = g_v[e, pl.ds(128 + o, 16)]
                v2 = g_v[e, pl.ds(160 + o, 16)]
                msg_v[e, pl.ds(32 + o, 16)] = v0 * gate + s0 * dirv
                msg_v[e, pl.ds(64 + o, 16)] = v1 * gate + s1 * dirv
                msg_v[e, pl.ds(96 + o, 16)] = v2 * gate + s2 * dirv

    def _body(jc, b, p, last):
        if not last:
            @pl.when(jc + 3 < CPT)
            def _():
                _issue_lin(jc + 3, (b + 3) % NBUF, p)

            @pl.when(jc + 2 < CPT)
            def _():
                _wait_lin((b + 2) % NBUF)
                _issue_g(jc + 2, (b + 2) % NBUF, p)
        _wait_g(b)

        @pl.when(jc >= NBUF)
        def _():
            _wait_sc(b)

        _compute(b)
        srci = plsc.load_gather(r_vs[b], [iota2])
        pltpu.async_copy(m_vs[b], slab.at[srci], ssems[b], add=True)

    @pl.loop(0, NPASS)
    def _pass(p):
        zv = jnp.zeros((16,), jnp.float32)
        for r in range(K):
            for h in range(8):
                m_vs[0][r, pl.ds(16 * h, 16)] = zv

        @pl.loop(0, ROWS_PT // K)
        def _zero(kk):
            pltpu.sync_copy(m_vs[0], slab.at[pl.ds(row0 + kk * K, K)])

        plsc.subcore_barrier()

        _issue_lin(0, 0, p)
        _issue_lin(1, 1, p)
        _issue_lin(2, 2, p)
        _wait_lin(0)
        _issue_g(0, 0, p)
        _wait_lin(1)
        _issue_g(1, 1, p)

        @pl.loop(0, CPT - 1, step=NBUF)
        def _chunk4(j):
            for t in range(NBUF):
                _body(j + t, t, p, False)

        _body(CPT - 1, (CPT - 1) % NBUF, p, True)

        for b in range(NBUF):
            _wait_sc(b)

        plsc.subcore_barrier()
        pltpu.sync_copy(slab.at[pl.ds(row0, ROWS_PT)],
                        out_hbm.at[cid, p, pl.ds(row0, ROWS_PT)])


_BN3 = 2000


def _k3_body(ns_ref, nv_ref, s_ref, os_ref, ov_ref):
    s = s_ref[...]
    agg = s[0] + s[1]
    for p in range(NPASS):
        os_ref[:, 32 * p:32 * p + 32] = (
            ns_ref[:, 32 * p:32 * p + 32] + 2.0 * agg[p, :, 0:32])
        for d in range(3):
            ov_ref[:, d, 32 * p:32 * p + 32] = (
                nv_ref[:, d, 32 * p:32 * p + 32]
                + 2.0 * agg[p, :, 32 + 32 * d:64 + 32 * d])


def _combine(node_scalars, node_vectors, S):
    return pl.pallas_call(
        _k3_body,
        grid=(N // _BN3,),
        in_specs=[
            pl.BlockSpec((_BN3, NODE), lambda i: (i, 0)),
            pl.BlockSpec((_BN3, 3, NODE), lambda i: (i, 0, 0)),
            pl.BlockSpec((2, NPASS, _BN3, 128), lambda i: (0, 0, i, 0)),
        ],
        out_specs=[
            pl.BlockSpec((_BN3, NODE), lambda i: (i, 0)),
            pl.BlockSpec((_BN3, 3, NODE), lambda i: (i, 0, 0)),
        ],
        out_shape=[
            jax.ShapeDtypeStruct((N, NODE), jnp.float32),
            jax.ShapeDtypeStruct((N, 3, NODE), jnp.float32),
        ],
    )(node_scalars, node_vectors, S)


def kernel(node_scalars, node_vectors, graph, edges_dist, edges_sense,
           W1, b1, W2, b2, We, be):
    perm = jnp.asarray(_PERM)
    W2p = W2[:, perm]
    b2p = b2[perm][None, :]
    Wep = We[:, perm].reshape(RBF_DIM, NPASS, 96).transpose(1, 0, 2)
    bep = be[perm].reshape(NPASS, 1, 96)
    Wext = jnp.concatenate(
        [Wep, bep, jnp.zeros((NPASS, 3, 96), jnp.float32)], axis=1)
    Wext = jnp.pad(Wext, ((0, 0), (0, 0), (0, 32)))

    G = _build_node_table(node_scalars, node_vectors, W1, b1[None, :],
                          W2p, b2p)
    G = G.reshape(NPASS * NPAD, 256)

    dist2d = edges_dist.reshape(_EROW, 128)
    F = _build_basis(dist2d).reshape(24, E)
    D = _build_edge_filter(F, edges_sense, Wext)

    gflat = graph.reshape(2 * E)
    S = _sc_edges(G, D, gflat)

    out_s, out_v = _combine(node_scalars, node_vectors, S)
    return (out_s, out_v)


# speedup vs baseline: 1.6117x; 1.6117x over previous
import functools

import numpy as np
import jax
import jax.numpy as jnp
from jax import lax
from jax.experimental import pallas as pl
from jax.experimental.pallas import tpu as pltpu
from jax.experimental.pallas import tpu_sc as plsc

R_CUT = 5.0
RBF_DIM = 20
NODE = 128
N = 10000
NPAD = 10240
E = 320000
NPASS = 4
K = 16
NBUF = 3
TILES = 32
EPT = E // TILES
CPT = EPT // K
ROWS_PT = NPAD // 16

_PERM = np.concatenate(
    [np.arange(128 * b + 32 * p, 128 * b + 32 * p + 32)
     for p in range(NPASS) for b in range(3)]
)


_BN1 = 1000


def _k1_body(ns_ref, nv_ref, w1_ref, b1_ref, w2p_ref, b2p_ref, g_ref):
    h = jnp.dot(ns_ref[...], w1_ref[...], preferred_element_type=jnp.float32)
    h = h + b1_ref[...]
    h = h * jax.nn.sigmoid(h)
    a = jnp.dot(h, w2p_ref[...], preferred_element_type=jnp.float32)
    a = a + b2p_ref[...]
    nv = nv_ref[...]
    for p in range(NPASS):
        g_ref[p, :, 0:96] = a[:, 96 * p:96 * p + 96]
        for d in range(3):
            g_ref[p, :, 96 + 32 * d:128 + 32 * d] = nv[:, d, 32 * p:32 * p + 32]
        g_ref[p, :, 192:256] = jnp.zeros((_BN1, 64), jnp.float32)


def _build_node_table(ns, nv, W1, b1, W2p, b2p):
    return pl.pallas_call(
        _k1_body,
        grid=(N // _BN1,),
        in_specs=[
            pl.BlockSpec((_BN1, NODE), lambda i: (i, 0)),
            pl.BlockSpec((_BN1, 3, NODE), lambda i: (i, 0, 0)),
            pl.BlockSpec((NODE, NODE), lambda i: (0, 0)),
            pl.BlockSpec((1, NODE), lambda i: (0, 0)),
            pl.BlockSpec((NODE, 3 * NODE), lambda i: (0, 0)),
            pl.BlockSpec((1, 3 * NODE), lambda i: (0, 0)),
        ],
        out_specs=pl.BlockSpec((NPASS, _BN1, 256), lambda i: (0, i, 0)),
        out_shape=jax.ShapeDtypeStruct((NPASS, NPAD, 256), jnp.float32),
    )(ns, nv, W1, b1, W2p, b2p)


_ROWS_A = 2500
_EROW = E // 128


def _k2a_body(d_ref, f_ref):
    d = d_ref[...]
    x = (jnp.pi / R_CUT) * d
    s1 = jnp.sin(x)
    c1 = jnp.cos(x)
    cut = 0.5 * c1 + 0.5
    g = cut / d
    two_c = 2.0 * c1
    s_prev = jnp.zeros_like(s1)
    s_cur = s1
    for k in range(RBF_DIM):
        f_ref[k, :, :] = s_cur * g
        s_next = two_c * s_cur - s_prev
        s_prev = s_cur
        s_cur = s_next
    f_ref[RBF_DIM, :, :] = cut
    z = jnp.zeros_like(s1)
    f_ref[RBF_DIM + 1, :, :] = z
    f_ref[RBF_DIM + 2, :, :] = z
    f_ref[RBF_DIM + 3, :, :] = z


def _build_basis(dist2d):
    return pl.pallas_call(
        _k2a_body,
        grid=(_EROW // _ROWS_A,),
        in_specs=[pl.BlockSpec((_ROWS_A, 128), lambda i: (i, 0))],
        out_specs=pl.BlockSpec((24, _ROWS_A, 128), lambda i: (0, i, 0)),
        out_shape=jax.ShapeDtypeStruct((24, _EROW, 128), jnp.float32),
    )(dist2d)


_BE2 = 3200


def _k2b_body(f_ref, sp_ref, wext_ref, out_ref):
    f = f_ref[...]
    sp = sp_ref[...]
    spz = jnp.concatenate(
        [jnp.zeros((_BE2, 96), jnp.float32), sp,
         jnp.zeros((_BE2, 29), jnp.float32)], axis=1)
    for p in range(NPASS):
        tp = lax.dot_general(
            f, wext_ref[p],
            dimension_numbers=(((0,), (0,)), ((), ())),
            preferred_element_type=jnp.float32)
        out_ref[p, :, :] = tp + spz


def _build_edge_filter(F, edges_sense, Wext):
    return pl.pallas_call(
        _k2b_body,
        grid=(E // _BE2,),
        in_specs=[
            pl.BlockSpec((24, _BE2), lambda i: (0, i)),
            pl.BlockSpec((_BE2, 3), lambda i: (i, 0)),
            pl.BlockSpec((NPASS, 24, 128), lambda i: (0, 0, 0)),
        ],
        out_specs=pl.BlockSpec((NPASS, _BE2, 128), lambda i: (0, i, 0)),
        out_shape=jax.ShapeDtypeStruct((NPASS, E, 128), jnp.float32),
    )(F, edges_sense, Wext)


_sc_mesh = plsc.VectorSubcoreMesh(
    core_axis_name="c", subcore_axis_name="s", num_cores=2)


@functools.partial(
    pl.kernel,
    out_type=jax.ShapeDtypeStruct((2, NPASS, NPAD, 128), jnp.float32),
    mesh=_sc_mesh,
    compiler_params=pltpu.CompilerParams(needs_layout_passes=False),
    scratch_types=(
        [pltpu.VMEM((EPT,), jnp.int32) for _ in range(2)]
        + [pltpu.VMEM((K, 256), jnp.float32) for _ in range(NBUF)]
        + [pltpu.VMEM((K, 128), jnp.float32) for _ in range(NBUF)]
        + [pltpu.VMEM((K, 128), jnp.float32) for _ in range(NBUF)]
        + [pltpu.VMEM_SHARED((NPAD, 128), jnp.float32)]
        + [pltpu.SemaphoreType.DMA for _ in range(3 * NBUF)]
    ),
)
def _sc_edges(g_hbm, d_hbm, src_hbm, dst_hbm, out_hbm, *scr):
    src_v, dst_v = scr[0], scr[1]
    g_vs = scr[2:2 + NBUF]
    d_vs = scr[2 + NBUF:2 + 2 * NBUF]
    m_vs = scr[2 + 2 * NBUF:2 + 3 * NBUF]
    slab = scr[2 + 3 * NBUF]
    sems = scr[3 + 3 * NBUF:]
    gsems = sems[0:NBUF]
    dsems = sems[NBUF:2 * NBUF]
    ssems = sems[2 * NBUF:3 * NBUF]

    cid = lax.axis_index("c")
    sid = lax.axis_index("s")
    wid = cid * 16 + sid
    row0 = sid * ROWS_PT

    pltpu.sync_copy(src_hbm.at[wid], src_v)
    pltpu.sync_copy(dst_hbm.at[wid], dst_v)

    def _issue(jc, b, p):
        gidx = dst_v[pl.ds(jc * K, K)] + p * NPAD
        pltpu.async_copy(g_hbm.at[gidx], g_vs[b], gsems[b])
        e0 = wid * EPT + jc * K
        pltpu.async_copy(d_hbm.at[p, pl.ds(e0, K)], d_vs[b], dsems[b])

    def _wait_in(b):
        pltpu.make_async_copy(g_hbm.at[pl.ds(0, K)], g_vs[b], gsems[b]).wait()
        pltpu.make_async_copy(d_hbm.at[0, pl.ds(0, K)], d_vs[b], dsems[b]).wait()

    def _wait_sc(b):
        pltpu.make_async_copy(m_vs[b], slab.at[pl.ds(0, K)], ssems[b]).wait()

    def _compute(b):
        g_v = g_vs[b]
        d_v = d_vs[b]
        msg_v = m_vs[b]

        @pl.loop(0, K)
        def _edge(e):
            sv = d_v[e, pl.ds(96, 16)]
            s0 = sv[0]
            s1 = sv[1]
            s2 = sv[2]
            for h in range(2):
                o = 16 * h
                a0 = g_v[e, pl.ds(o, 16)]
                a1 = g_v[e, pl.ds(32 + o, 16)]
                a2 = g_v[e, pl.ds(64 + o, 16)]
                dd0 = d_v[e, pl.ds(o, 16)]
                dd1 = d_v[e, pl.ds(32 + o, 16)]
                dd2 = d_v[e, pl.ds(64 + o, 16)]
                gate = a0 * dd0
                dirv = a2 * dd2
                msg_v[e, pl.ds(o, 16)] = a1 * dd1
                v0 = g_v[e, pl.ds(96 + o, 16)]
                v1 = g_v[e, pl.ds(128 + o, 16)]
                v2 = g_v[e, pl.ds(160 + o, 16)]
                msg_v[e, pl.ds(32 + o, 16)] = v0 * gate + s0 * dirv
                msg_v[e, pl.ds(64 + o, 16)] = v1 * gate + s1 * dirv
                msg_v[e, pl.ds(96 + o, 16)] = v2 * gate + s2 * dirv

    def _body(jc, b, p, last):
        if not last:
            @pl.when(jc + 2 < CPT)
            def _():
                _issue(jc + 2, (b + 2) % NBUF, p)
        _wait_in(b)

        @pl.when(jc >= NBUF)
        def _():
            _wait_sc(b)

        _compute(b)
        srci = src_v[pl.ds(jc * K, K)]
        pltpu.async_copy(m_vs[b], slab.at[srci], ssems[b], add=True)

    @pl.loop(0, NPASS)
    def _pass(p):
        zv = jnp.zeros((16,), jnp.float32)
        for r in range(K):
            for h in range(8):
                m_vs[0][r, pl.ds(16 * h, 16)] = zv

        @pl.loop(0, ROWS_PT // K)
        def _zero(kk):
            pltpu.sync_copy(m_vs[0], slab.at[pl.ds(row0 + kk * K, K)])

        plsc.subcore_barrier()

        _issue(0, 0, p)
        _issue(1, 1, p)

        @pl.loop(0, CPT - 1, step=NBUF)
        def _chunk3(j):
            for t in range(NBUF):
                _body(j + t, t, p, False)

        _body(CPT - 1, (CPT - 1) % NBUF, p, True)

        for b in range(NBUF):
            _wait_sc(b)

        plsc.subcore_barrier()
        pltpu.sync_copy(slab.at[pl.ds(row0, ROWS_PT)],
                        out_hbm.at[cid, p, pl.ds(row0, ROWS_PT)])


_BN3 = 2000


def _k3_body(ns_ref, nv_ref, s_ref, os_ref, ov_ref):
    s = s_ref[...]
    agg = s[0] + s[1]
    for p in range(NPASS):
        os_ref[:, 32 * p:32 * p + 32] = (
            ns_ref[:, 32 * p:32 * p + 32] + 2.0 * agg[p, :, 0:32])
        for d in range(3):
            ov_ref[:, d, 32 * p:32 * p + 32] = (
                nv_ref[:, d, 32 * p:32 * p + 32]
                + 2.0 * agg[p, :, 32 + 32 * d:64 + 32 * d])


def _combine(node_scalars, node_vectors, S):
    return pl.pallas_call(
        _k3_body,
        grid=(N // _BN3,),
        in_specs=[
            pl.BlockSpec((_BN3, NODE), lambda i: (i, 0)),
            pl.BlockSpec((_BN3, 3, NODE), lambda i: (i, 0, 0)),
            pl.BlockSpec((2, NPASS, _BN3, 128), lambda i: (0, 0, i, 0)),
        ],
        out_specs=[
            pl.BlockSpec((_BN3, NODE), lambda i: (i, 0)),
            pl.BlockSpec((_BN3, 3, NODE), lambda i: (i, 0, 0)),
        ],
        out_shape=[
            jax.ShapeDtypeStruct((N, NODE), jnp.float32),
            jax.ShapeDtypeStruct((N, 3, NODE), jnp.float32),
        ],
    )(node_scalars, node_vectors, S)


def kernel(node_scalars, node_vectors, graph, edges_dist, edges_sense,
           W1, b1, W2, b2, We, be):
    perm = jnp.asarray(_PERM)
    W2p = W2[:, perm]
    b2p = b2[perm][None, :]
    Wep = We[:, perm].reshape(RBF_DIM, NPASS, 96).transpose(1, 0, 2)
    bep = be[perm].reshape(NPASS, 1, 96)
    Wext = jnp.concatenate(
        [Wep, bep, jnp.zeros((NPASS, 3, 96), jnp.float32)], axis=1)
    Wext = jnp.pad(Wext, ((0, 0), (0, 0), (0, 32)))

    G = _build_node_table(node_scalars, node_vectors, W1, b1[None, :],
                          W2p, b2p)
    G = G.reshape(NPASS * NPAD, 256)

    dist2d = edges_dist.reshape(_EROW, 128)
    F = _build_basis(dist2d).reshape(24, E)
    D = _build_edge_filter(F, edges_sense, Wext)

    src_ids = graph[:, 0].reshape(TILES, EPT)
    dst_ids = graph[:, 1].reshape(TILES, EPT)
    S = _sc_edges(G, D, src_ids, dst_ids)

    out_s, out_v = _combine(node_scalars, node_vectors, S)
    return (out_s, out_v)


# speedup vs baseline: 1.6133x; 1.0010x over previous
import functools

import numpy as np
import jax
import jax.numpy as jnp
from jax import lax
from jax.experimental import pallas as pl
from jax.experimental.pallas import tpu as pltpu
from jax.experimental.pallas import tpu_sc as plsc

R_CUT = 5.0
RBF_DIM = 20
NODE = 128
N = 10000
NPAD = 10240
E = 320000
NPASS = 4
K = 16
NBUF = 3
TILES = 32
EPT = E // TILES
CPT = EPT // K
ROWS_PT = NPAD // 16

_PERM = np.concatenate(
    [np.arange(128 * b + 32 * p, 128 * b + 32 * p + 32)
     for p in range(NPASS) for b in range(3)]
)


_BN1 = 1000


def _k1_body(ns_ref, nv_ref, w1_ref, b1_ref, w2p_ref, b2p_ref, g_ref):
    h = jnp.dot(ns_ref[...], w1_ref[...], preferred_element_type=jnp.float32)
    h = h + b1_ref[...]
    h = h * jax.nn.sigmoid(h)
    a = jnp.dot(h, w2p_ref[...], preferred_element_type=jnp.float32)
    a = a + b2p_ref[...]
    nv = nv_ref[...]
    for p in range(NPASS):
        g_ref[p, :, 0:96] = a[:, 96 * p:96 * p + 96]
        for d in range(3):
            g_ref[p, :, 96 + 32 * d:128 + 32 * d] = nv[:, d, 32 * p:32 * p + 32]
        g_ref[p, :, 192:256] = jnp.zeros((_BN1, 64), jnp.float32)


def _build_node_table(ns, nv, W1, b1, W2p, b2p):
    return pl.pallas_call(
        _k1_body,
        grid=(N // _BN1,),
        in_specs=[
            pl.BlockSpec((_BN1, NODE), lambda i: (i, 0)),
            pl.BlockSpec((_BN1, 3, NODE), lambda i: (i, 0, 0)),
            pl.BlockSpec((NODE, NODE), lambda i: (0, 0)),
            pl.BlockSpec((1, NODE), lambda i: (0, 0)),
            pl.BlockSpec((NODE, 3 * NODE), lambda i: (0, 0)),
            pl.BlockSpec((1, 3 * NODE), lambda i: (0, 0)),
        ],
        out_specs=pl.BlockSpec((NPASS, _BN1, 256), lambda i: (0, i, 0)),
        out_shape=jax.ShapeDtypeStruct((NPASS, NPAD, 256), jnp.float32),
    )(ns, nv, W1, b1, W2p, b2p)


_ROWS_A = 2500
_EROW = E // 128


def _k2a_body(d_ref, f_ref):
    d = d_ref[...]
    x = (jnp.pi / R_CUT) * d
    s1 = jnp.sin(x)
    c1 = jnp.cos(x)
    cut = 0.5 * c1 + 0.5
    g = cut / d
    two_c = 2.0 * c1
    s_prev = jnp.zeros_like(s1)
    s_cur = s1
    for k in range(RBF_DIM):
        f_ref[k, :, :] = s_cur * g
        s_next = two_c * s_cur - s_prev
        s_prev = s_cur
        s_cur = s_next
    f_ref[RBF_DIM, :, :] = cut
    z = jnp.zeros_like(s1)
    f_ref[RBF_DIM + 1, :, :] = z
    f_ref[RBF_DIM + 2, :, :] = z
    f_ref[RBF_DIM + 3, :, :] = z


def _build_basis(dist2d):
    return pl.pallas_call(
        _k2a_body,
        grid=(_EROW // _ROWS_A,),
        in_specs=[pl.BlockSpec((_ROWS_A, 128), lambda i: (i, 0))],
        out_specs=pl.BlockSpec((24, _ROWS_A, 128), lambda i: (0, i, 0)),
        out_shape=jax.ShapeDtypeStruct((24, _EROW, 128), jnp.float32),
    )(dist2d)


_BE2 = 3200


def _k2b_body(f_ref, sp_ref, wext_ref, out_ref):
    f = f_ref[...]
    sp = sp_ref[...]
    spz = jnp.concatenate(
        [jnp.zeros((_BE2, 96), jnp.float32), sp,
         jnp.zeros((_BE2, 29), jnp.float32)], axis=1)
    for p in range(NPASS):
        tp = lax.dot_general(
            f, wext_ref[p],
            dimension_numbers=(((0,), (0,)), ((), ())),
            preferred_element_type=jnp.float32)
        out_ref[p, :, :] = tp + spz


def _build_edge_filter(F, edges_sense, Wext):
    return pl.pallas_call(
        _k2b_body,
        grid=(E // _BE2,),
        in_specs=[
            pl.BlockSpec((24, _BE2), lambda i: (0, i)),
            pl.BlockSpec((_BE2, 3), lambda i: (i, 0)),
            pl.BlockSpec((NPASS, 24, 128), lambda i: (0, 0, 0)),
        ],
        out_specs=pl.BlockSpec((NPASS, _BE2, 128), lambda i: (0, i, 0)),
        out_shape=jax.ShapeDtypeStruct((NPASS, E, 128), jnp.float32),
    )(F, edges_sense, Wext)


_sc_mesh = plsc.VectorSubcoreMesh(
    core_axis_name="c", subcore_axis_name="s", num_cores=2)


@functools.partial(
    pl.kernel,
    out_type=jax.ShapeDtypeStruct((2, NPASS, NPAD, 128), jnp.float32),
    mesh=_sc_mesh,
    compiler_params=pltpu.CompilerParams(needs_layout_passes=False),
    scratch_types=(
        [pltpu.VMEM((EPT,), jnp.int32) for _ in range(2)]
        + [pltpu.VMEM((K, 256), jnp.float32) for _ in range(NBUF)]
        + [pltpu.VMEM((K, 128), jnp.float32) for _ in range(NBUF)]
        + [pltpu.VMEM((K, 128), jnp.float32) for _ in range(NBUF)]
        + [pltpu.VMEM_SHARED((NPAD, 128), jnp.float32)]
        + [pltpu.SemaphoreType.DMA for _ in range(3 * NBUF)]
    ),
)
def _sc_edges(g_hbm, d_hbm, src_hbm, dst_hbm, out_hbm, *scr):
    src_v, dst_v = scr[0], scr[1]
    g_vs = scr[2:2 + NBUF]
    d_vs = scr[2 + NBUF:2 + 2 * NBUF]
    m_vs = scr[2 + 2 * NBUF:2 + 3 * NBUF]
    slab = scr[2 + 3 * NBUF]
    sems = scr[3 + 3 * NBUF:]
    gsems = sems[0:NBUF]
    dsems = sems[NBUF:2 * NBUF]
    ssems = sems[2 * NBUF:3 * NBUF]

    cid = lax.axis_index("c")
    sid = lax.axis_index("s")
    wid = cid * 16 + sid
    row0 = sid * ROWS_PT

    pltpu.sync_copy(src_hbm.at[wid], src_v)
    pltpu.sync_copy(dst_hbm.at[wid], dst_v)

    def _issue(jc, b, p):
        gidx = dst_v[pl.ds(jc * K, K)] + p * NPAD
        pltpu.async_copy(g_hbm.at[gidx], g_vs[b], gsems[b])
        e0 = wid * EPT + jc * K
        pltpu.async_copy(d_hbm.at[p, pl.ds(e0, K)], d_vs[b], dsems[b])

    def _wait_in(b):
        pltpu.make_async_copy(g_hbm.at[pl.ds(0, K)], g_vs[b], gsems[b]).wait()
        pltpu.make_async_copy(d_hbm.at[0, pl.ds(0, K)], d_vs[b], dsems[b]).wait()

    def _wait_sc(b):
        pltpu.make_async_copy(m_vs[b], slab.at[pl.ds(0, K)], ssems[b]).wait()

    def _compute(b):
        g_v = g_vs[b]
        d_v = d_vs[b]
        msg_v = m_vs[b]

        @pl.loop(0, K, unroll=4)
        def _edge(e):
            sv = d_v[e, pl.ds(96, 16)]
            s0 = sv[0]
            s1 = sv[1]
            s2 = sv[2]
            for h in range(2):
                o = 16 * h
                a0 = g_v[e, pl.ds(o, 16)]
                a1 = g_v[e, pl.ds(32 + o, 16)]
                a2 = g_v[e, pl.ds(64 + o, 16)]
                dd0 = d_v[e, pl.ds(o, 16)]
                dd1 = d_v[e, pl.ds(32 + o, 16)]
                dd2 = d_v[e, pl.ds(64 + o, 16)]
                gate = a0 * dd0
                dirv = a2 * dd2
                msg_v[e, pl.ds(o, 16)] = a1 * dd1
                v0 = g_v[e, pl.ds(96 + o, 16)]
                v1 = g_v[e, pl.ds(128 + o, 16)]
                v2 = g_v[e, pl.ds(160 + o, 16)]
                msg_v[e, pl.ds(32 + o, 16)] = v0 * gate + s0 * dirv
                msg_v[e, pl.ds(64 + o, 16)] = v1 * gate + s1 * dirv
                msg_v[e, pl.ds(96 + o, 16)] = v2 * gate + s2 * dirv

    def _body(jc, b, p, last):
        if not last:
            @pl.when(jc + 2 < CPT)
            def _():
                _issue(jc + 2, (b + 2) % NBUF, p)
        _wait_in(b)

        @pl.when(jc >= NBUF)
        def _():
            _wait_sc(b)

        _compute(b)
        srci = src_v[pl.ds(jc * K, K)]
        pltpu.async_copy(m_vs[b], slab.at[srci], ssems[b], add=True)

    @pl.loop(0, NPASS)
    def _pass(p):
        zv = jnp.zeros((16,), jnp.float32)
        for r in range(K):
            for h in range(8):
                m_vs[0][r, pl.ds(16 * h, 16)] = zv

        @pl.loop(0, ROWS_PT // K)
        def _zero(kk):
            pltpu.sync_copy(m_vs[0], slab.at[pl.ds(row0 + kk * K, K)])

        plsc.subcore_barrier()

        _issue(0, 0, p)
        _issue(1, 1, p)

        @pl.loop(0, CPT - 1, step=NBUF)
        def _chunk3(j):
            for t in range(NBUF):
                _body(j + t, t, p, False)

        _body(CPT - 1, (CPT - 1) % NBUF, p, True)

        for b in range(NBUF):
            _wait_sc(b)

        plsc.subcore_barrier()
        pltpu.sync_copy(slab.at[pl.ds(row0, ROWS_PT)],
                        out_hbm.at[cid, p, pl.ds(row0, ROWS_PT)])


_BN3 = 2000


def _k3_body(ns_ref, nv_ref, s_ref, os_ref, ov_ref):
    s = s_ref[...]
    agg = s[0] + s[1]
    for p in range(NPASS):
        os_ref[:, 32 * p:32 * p + 32] = (
            ns_ref[:, 32 * p:32 * p + 32] + 2.0 * agg[p, :, 0:32])
        for d in range(3):
            ov_ref[:, d, 32 * p:32 * p + 32] = (
                nv_ref[:, d, 32 * p:32 * p + 32]
                + 2.0 * agg[p, :, 32 + 32 * d:64 + 32 * d])


def _combine(node_scalars, node_vectors, S):
    return pl.pallas_call(
        _k3_body,
        grid=(N // _BN3,),
        in_specs=[
            pl.BlockSpec((_BN3, NODE), lambda i: (i, 0)),
            pl.BlockSpec((_BN3, 3, NODE), lambda i: (i, 0, 0)),
            pl.BlockSpec((2, NPASS, _BN3, 128), lambda i: (0, 0, i, 0)),
        ],
        out_specs=[
            pl.BlockSpec((_BN3, NODE), lambda i: (i, 0)),
            pl.BlockSpec((_BN3, 3, NODE), lambda i: (i, 0, 0)),
        ],
        out_shape=[
            jax.ShapeDtypeStruct((N, NODE), jnp.float32),
            jax.ShapeDtypeStruct((N, 3, NODE), jnp.float32),
        ],
    )(node_scalars, node_vectors, S)


def kernel(node_scalars, node_vectors, graph, edges_dist, edges_sense,
           W1, b1, W2, b2, We, be):
    perm = jnp.asarray(_PERM)
    W2p = W2[:, perm]
    b2p = b2[perm][None, :]
    Wep = We[:, perm].reshape(RBF_DIM, NPASS, 96).transpose(1, 0, 2)
    bep = be[perm].reshape(NPASS, 1, 96)
    Wext = jnp.concatenate(
        [Wep, bep, jnp.zeros((NPASS, 3, 96), jnp.float32)], axis=1)
    Wext = jnp.pad(Wext, ((0, 0), (0, 0), (0, 32)))

    G = _build_node_table(node_scalars, node_vectors, W1, b1[None, :],
                          W2p, b2p)
    G = G.reshape(NPASS * NPAD, 256)

    dist2d = edges_dist.reshape(_EROW, 128)
    F = _build_basis(dist2d).reshape(24, E)
    D = _build_edge_filter(F, edges_sense, Wext)

    src_ids = graph[:, 0].reshape(TILES, EPT)
    dst_ids = graph[:, 1].reshape(TILES, EPT)
    S = _sc_edges(G, D, src_ids, dst_ids)

    out_s, out_v = _combine(node_scalars, node_vectors, S)
    return (out_s, out_v)
